# Initial kernel scaffold; baseline (speedup 1.0000x reference)
import jax
import jax.numpy as jnp
from jax.experimental import pallas as pl


def kernel(user_table, item_table, attr_table, user_nodes, item_nodes, attribute_nodes):
    raise NotImplementedError("write your pallas kernel here")


# speedup vs baseline: 1.6270x; 1.6270x over previous
import functools

import jax
import jax.numpy as jnp
from jax import lax
from jax.experimental import pallas as pl
from jax.experimental.pallas import tpu as pltpu
from jax.experimental.pallas import tpu_sc as plsc

_B = 4096
_DU = 128
_DA = 32


@functools.lru_cache(maxsize=None)
def _build():
    info = plsc.get_sparse_core_info()
    nc, ns = info.num_cores, info.num_subcores
    nw = nc * ns
    bpw = _B // nw

    mesh = plsc.VectorSubcoreMesh(core_axis_name="c", subcore_axis_name="s")

    @functools.partial(
        pl.kernel,
        mesh=mesh,
        out_type=(
            jax.ShapeDtypeStruct((_B, _DU), jnp.float32),
            jax.ShapeDtypeStruct((_B, _DU), jnp.float32),
            jax.ShapeDtypeStruct((_B, _DU), jnp.float32),
        ),
        scratch_types=[
            pltpu.VMEM((bpw,), jnp.int32),
            pltpu.VMEM((bpw,), jnp.int32),
            pltpu.VMEM((bpw,), jnp.int32),
            pltpu.VMEM((bpw, _DU), jnp.float32),
            pltpu.VMEM((bpw, _DU), jnp.float32),
            pltpu.VMEM((bpw, _DU), jnp.float32),
            pltpu.SemaphoreType.DMA,
            pltpu.SemaphoreType.DMA,
            pltpu.SemaphoreType.DMA,
        ],
    )
    def emb(user_hbm, item_hbm, attr_hbm, un_hbm, in_hbm, an_hbm,
            u_out, i_out, a_out,
            uidx, iidx, aidx, urows, irows, arows, su, si, sa):
        wid = lax.axis_index("s") * nc + lax.axis_index("c")
        base = wid * bpw
        pltpu.sync_copy(un_hbm.at[pl.ds(base, bpw)], uidx)
        pltpu.sync_copy(in_hbm.at[pl.ds(base, bpw)], iidx)
        pltpu.sync_copy(an_hbm.at[pl.ds(base, bpw)], aidx)
        cu = pltpu.async_copy(user_hbm.at[uidx], urows, su)
        ci = pltpu.async_copy(item_hbm.at[iidx], irows, si)
        ca = pltpu.async_copy(attr_hbm.at[aidx], arows, sa)
        cu.wait()
        pltpu.sync_copy(urows, u_out.at[pl.ds(base, bpw)])
        ci.wait()
        pltpu.sync_copy(irows, i_out.at[pl.ds(base, bpw)])
        ca.wait()
        pltpu.sync_copy(arows, a_out.at[pl.ds(base, bpw)])

    return emb


def kernel(user_table, item_table, attr_table, user_nodes, item_nodes,
           attribute_nodes):
    emb = _build()
    attr_padded = jnp.pad(attr_table, ((0, 0), (0, _DU - _DA)))
    u, i, a = emb(
        user_table, item_table, attr_padded,
        user_nodes.astype(jnp.int32),
        item_nodes.astype(jnp.int32),
        attribute_nodes.astype(jnp.int32),
    )
    return (u, i, a[:, :_DA])
